# baseline (speedup 1.0000x reference)
import functools

import jax
import jax.numpy as jnp
from jax import lax
from jax.experimental import pallas as pl
from jax.experimental.pallas import tpu as pltpu
from jax.experimental.pallas import tpu_sc as plsc

N = 10000
K = 32
D = 128
DW = D // 2
DE = 16

NC = 2
NS = 16
NW = NC * NS

PW0 = 352
PW1 = 288
CUT = NS * PW0
NP = NS * (PW0 + PW1)
NIP = CUT + 15 * PW1 + PW0
CH = 4
RK = CH * K
NCH0 = PW0 // CH
NCH1 = PW1 // CH
NBUF = 8
NOBUF = 2

_DOT11 = (((1,), (1,)), ((), ()))


def _sc_gather_sum_body(idx_hbm, xp_hbm, out_hbm,
                        idx_all, xsp, b0, b1, b2, b3, b4, b5, b6, b7,
                        ob0, ob1, sg0, sg1, sg2, sg3, sg4, sg5, sg6, sg7,
                        so0, so1):
    cid = lax.axis_index("c")
    sid = lax.axis_index("s")
    cbase = jnp.where(cid == 0, sid * PW0, CUT + sid * PW1)
    nchunk = jnp.where(cid == 0, NCH0, NCH1)

    bufs = (b0, b1, b2, b3, b4, b5, b6, b7)
    gsems = (sg0, sg1, sg2, sg3, sg4, sg5, sg6, sg7)
    obufs = (ob0, ob1)
    osems = (so0, so1)

    rows_per_sub = N // NS
    pltpu.sync_copy(xp_hbm.at[pl.ds(sid * rows_per_sub, rows_per_sub)],
                    xsp.at[pl.ds(sid * rows_per_sub, rows_per_sub)])
    pltpu.sync_copy(idx_hbm.at[pl.ds(cbase * K, PW0 * K)], idx_all)
    plsc.subcore_barrier()

    def gather_cps(c, p):
        return [pltpu.make_async_copy(
            xsp.at[idx_all.at[pl.ds(c * RK, RK)]], bufs[p], gsems[p])]

    def out_cp(c, t):
        return pltpu.make_async_copy(
            obufs[t], out_hbm.at[pl.ds(cbase + c * CH, CH)], osems[t])

    for p in range(NBUF):
        for cp in gather_cps(p, p):
            cp.start()

    def ring(i, carry):
        for p in range(NBUF):
            c = i * NBUF + p
            t = p % NOBUF
            for cp in gather_cps(c, p):
                cp.wait()

            @pl.when(c >= NOBUF)
            def _():
                out_cp(c - NOBUF, t).wait()

            buf = bufs[p]
            obuf = obufs[t]

            def center(g, carry2):
                row = g * K
                for d in range(4):
                    accs = [None, None, None, None]
                    for k in range(K):
                        v = buf[row + k, pl.ds(d * 16, 16)]
                        fe = plsc.bitcast(v << 16, jnp.float32)
                        fo = plsc.bitcast(v, jnp.float32)
                        h = k & 1
                        accs[h] = fe if accs[h] is None else accs[h] + fe
                        accs[2 + h] = fo if accs[2 + h] is None \
                            else accs[2 + h] + fo
                    obuf[g, pl.ds(d * 16, 16)] = accs[0] + accs[1]
                    obuf[g, pl.ds(64 + d * 16, 16)] = accs[2] + accs[3]
                return carry2

            lax.fori_loop(0, CH, center, 0)
            out_cp(c, t).start()

            @pl.when(c + NBUF < nchunk)
            def _():
                for cp in gather_cps(c + NBUF, p):
                    cp.start()
        return carry

    lax.fori_loop(0, nchunk // NBUF, ring, 0)
    out_cp(nchunk - 2, 0).wait()
    out_cp(nchunk - 1, 1).wait()


def _sc_gather_sum(idx_flat, xp):
    mesh = plsc.VectorSubcoreMesh(core_axis_name="c", subcore_axis_name="s")
    return pl.kernel(
        _sc_gather_sum_body,
        mesh=mesh,
        compiler_params=pltpu.CompilerParams(
            needs_layout_passes=False, use_tc_tiling_on_sc=False),
        out_type=jax.ShapeDtypeStruct((NP, D), jnp.float32),
        scratch_types=[
            pltpu.VMEM((PW0 * K,), jnp.int32),
            pltpu.VMEM_SHARED((N, DW), jnp.int32),
            pltpu.VMEM((RK, DW), jnp.int32),
            pltpu.VMEM((RK, DW), jnp.int32),
            pltpu.VMEM((RK, DW), jnp.int32),
            pltpu.VMEM((RK, DW), jnp.int32),
            pltpu.VMEM((RK, DW), jnp.int32),
            pltpu.VMEM((RK, DW), jnp.int32),
            pltpu.VMEM((RK, DW), jnp.int32),
            pltpu.VMEM((RK, DW), jnp.int32),
            pltpu.VMEM((CH, D), jnp.float32),
            pltpu.VMEM((CH, D), jnp.float32),
            pltpu.SemaphoreType.DMA,
            pltpu.SemaphoreType.DMA,
            pltpu.SemaphoreType.DMA,
            pltpu.SemaphoreType.DMA,
            pltpu.SemaphoreType.DMA,
            pltpu.SemaphoreType.DMA,
            pltpu.SemaphoreType.DMA,
            pltpu.SemaphoreType.DMA,
            pltpu.SemaphoreType.DMA,
            pltpu.SemaphoreType.DMA,
        ],
    )(idx_flat, xp)


def _pack_body(x_ref, o_ref):
    xb = x_ref[...]
    lo = xb[:, :DW].astype(jnp.bfloat16)
    hi = xb[:, DW:].astype(jnp.bfloat16)
    loi = lax.bitcast_convert_type(lo, jnp.uint16).astype(jnp.int32)
    hii = lax.bitcast_convert_type(hi, jnp.uint16).astype(jnp.int32)
    o_ref[...] = (hii << 16) | loi


def _pack(x):
    B = 2000
    return pl.pallas_call(
        _pack_body,
        grid=(N // B,),
        in_specs=[pl.BlockSpec((B, D), lambda i: (i, 0))],
        out_specs=pl.BlockSpec((B, DW), lambda i: (i, 0)),
        out_shape=jax.ShapeDtypeStruct((N, DW), jnp.int32),
    )(x)


def _t1_body(x_ref, ef_ref, wc_ref, we_ref, o_ref):
    y = lax.dot_general(x_ref[...], wc_ref[...], _DOT11,
                        preferred_element_type=jnp.float32)
    wbig = jnp.concatenate([we_ref[...]] * K, axis=1)
    y = y + lax.dot_general(ef_ref[...] * (1.0 / K), wbig, _DOT11,
                            preferred_element_type=jnp.float32)
    o_ref[...] = y


def _t1(x, e_flat, Wc, We):
    B = 2000
    return pl.pallas_call(
        _t1_body,
        grid=(N // B,),
        in_specs=[
            pl.BlockSpec((B, D), lambda i: (i, 0)),
            pl.BlockSpec((B, K * DE), lambda i: (i, 0)),
            pl.BlockSpec((D, D), lambda i: (0, 0)),
            pl.BlockSpec((D, DE), lambda i: (0, 0)),
        ],
        out_specs=pl.BlockSpec((B, D), lambda i: (i, 0)),
        out_shape=jax.ShapeDtypeStruct((N, D), jnp.float32),
    )(x, e_flat, Wc, We)


def _final_body(t1_ref, s_ref, wn_ref, o_ref):
    y = t1_ref[...] + lax.dot_general(
        s_ref[...] * (1.0 / K), wn_ref[...], _DOT11,
        preferred_element_type=jnp.float32)
    o_ref[...] = jnp.maximum(y, 0.0)


def _final(t1, s_pad, Wn):
    B = 2000
    return pl.pallas_call(
        _final_body,
        grid=(N // B,),
        in_specs=[
            pl.BlockSpec((B, D), lambda i: (i, 0)),
            pl.BlockSpec((B, D), lambda i: (i, 0)),
            pl.BlockSpec((D, D), lambda i: (0, 0)),
        ],
        out_specs=pl.BlockSpec((B, D), lambda i: (i, 0)),
        out_shape=jax.ShapeDtypeStruct((N, D), jnp.float32),
    )(t1, s_pad, Wn)


def kernel(x, e, ij, Wc, Wn, We):
    idx = ij.reshape(N * K)
    idx_pad = jnp.concatenate(
        [idx, jnp.zeros((NIP * K - N * K,), dtype=jnp.int32)])
    xp = _pack(x)
    s_pad = _sc_gather_sum(idx_pad, xp)
    t1 = _t1(x, e.reshape(N, K * DE), Wc, We)
    return _final(t1, s_pad, Wn)


# speedup vs baseline: 1.0395x; 1.0395x over previous
import functools

import jax
import jax.numpy as jnp
from jax import lax
from jax.experimental import pallas as pl
from jax.experimental.pallas import tpu as pltpu
from jax.experimental.pallas import tpu_sc as plsc

N = 10000
K = 32
D = 128
DW = D // 2
DE = 16

NC = 2
NS = 16
NW = NC * NS

PW0 = 352
PW1 = 288
CUT = NS * PW0
NP = NS * (PW0 + PW1)
NIP = CUT + 15 * PW1 + PW0
CH = 4
RK = CH * K
NCH0 = PW0 // CH
NCH1 = PW1 // CH
NBUF = 4
NOBUF = 2

_DOT11 = (((1,), (1,)), ((), ()))


def _sc_gather_sum_body(idx_hbm, xp_hbm, out_hbm,
                        idx_all, xsp, b0, b1, b2, b3, ob0, ob1,
                        sg0, sg1, sg2, sg3, so0, so1):
    cid = lax.axis_index("c")
    sid = lax.axis_index("s")
    cbase = jnp.where(cid == 0, sid * PW0, CUT + sid * PW1)
    nchunk = jnp.where(cid == 0, NCH0, NCH1)

    bufs = (b0, b1, b2, b3)
    gsems = (sg0, sg1, sg2, sg3)
    obufs = (ob0, ob1)
    osems = (so0, so1)

    rows_per_sub = N // NS
    pltpu.sync_copy(xp_hbm.at[pl.ds(sid * rows_per_sub, rows_per_sub)],
                    xsp.at[pl.ds(sid * rows_per_sub, rows_per_sub)])
    pltpu.sync_copy(idx_hbm.at[pl.ds(cbase * K, PW0 * K)], idx_all)
    plsc.subcore_barrier()

    def gather_cps(c, p):
        return [pltpu.make_async_copy(
            xsp.at[idx_all.at[pl.ds(c * RK, RK)]], bufs[p], gsems[p])]

    def out_cp(c, t):
        return pltpu.make_async_copy(
            obufs[t], out_hbm.at[pl.ds(cbase + c * CH, CH)], osems[t])

    for p in range(NBUF):
        for cp in gather_cps(p, p):
            cp.start()

    def ring(i, carry):
        for p in range(NBUF):
            c = i * NBUF + p
            t = p % NOBUF
            for cp in gather_cps(c, p):
                cp.wait()

            @pl.when(c >= NOBUF)
            def _():
                out_cp(c - NOBUF, t).wait()

            buf = bufs[p]
            obuf = obufs[t]

            def center(g, carry2):
                row = g * K
                for d in range(4):
                    accs = [None, None, None, None]
                    for k in range(K):
                        v = buf[row + k, pl.ds(d * 16, 16)]
                        fe = plsc.bitcast(v << 16, jnp.float32)
                        fo = plsc.bitcast(v, jnp.float32)
                        h = k & 1
                        accs[h] = fe if accs[h] is None else accs[h] + fe
                        accs[2 + h] = fo if accs[2 + h] is None \
                            else accs[2 + h] + fo
                    obuf[g, pl.ds(d * 16, 16)] = accs[0] + accs[1]
                    obuf[g, pl.ds(64 + d * 16, 16)] = accs[2] + accs[3]
                return carry2

            lax.fori_loop(0, CH, center, 0)
            out_cp(c, t).start()

            @pl.when(c + NBUF < nchunk)
            def _():
                for cp in gather_cps(c + NBUF, p):
                    cp.start()
        return carry

    lax.fori_loop(0, nchunk // NBUF, ring, 0)
    out_cp(nchunk - 2, 0).wait()
    out_cp(nchunk - 1, 1).wait()


def _sc_gather_sum(idx_flat, xp):
    mesh = plsc.VectorSubcoreMesh(core_axis_name="c", subcore_axis_name="s")
    return pl.kernel(
        _sc_gather_sum_body,
        mesh=mesh,
        compiler_params=pltpu.CompilerParams(
            needs_layout_passes=False, use_tc_tiling_on_sc=False),
        out_type=jax.ShapeDtypeStruct((NP, D), jnp.float32),
        scratch_types=[
            pltpu.VMEM((PW0 * K,), jnp.int32),
            pltpu.VMEM_SHARED((N, DW), jnp.int32),
            pltpu.VMEM((RK, DW), jnp.int32),
            pltpu.VMEM((RK, DW), jnp.int32),
            pltpu.VMEM((RK, DW), jnp.int32),
            pltpu.VMEM((RK, DW), jnp.int32),
            pltpu.VMEM((CH, D), jnp.float32),
            pltpu.VMEM((CH, D), jnp.float32),
            pltpu.SemaphoreType.DMA,
            pltpu.SemaphoreType.DMA,
            pltpu.SemaphoreType.DMA,
            pltpu.SemaphoreType.DMA,
            pltpu.SemaphoreType.DMA,
            pltpu.SemaphoreType.DMA,
        ],
    )(idx_flat, xp)


def _pack_body(x_ref, o_ref):
    xb = x_ref[...]
    lo = xb[:, :DW].astype(jnp.bfloat16)
    hi = xb[:, DW:].astype(jnp.bfloat16)
    loi = lax.bitcast_convert_type(lo, jnp.uint16).astype(jnp.int32)
    hii = lax.bitcast_convert_type(hi, jnp.uint16).astype(jnp.int32)
    o_ref[...] = (hii << 16) | loi


def _pack(x):
    B = 2000
    return pl.pallas_call(
        _pack_body,
        grid=(N // B,),
        in_specs=[pl.BlockSpec((B, D), lambda i: (i, 0))],
        out_specs=pl.BlockSpec((B, DW), lambda i: (i, 0)),
        out_shape=jax.ShapeDtypeStruct((N, DW), jnp.int32),
    )(x)


def _t1_body(x_ref, ef_ref, wc_ref, we_ref, o_ref):
    y = lax.dot_general(x_ref[...], wc_ref[...], _DOT11,
                        preferred_element_type=jnp.float32)
    wbig = jnp.concatenate([we_ref[...]] * K, axis=1)
    y = y + lax.dot_general(ef_ref[...] * (1.0 / K), wbig, _DOT11,
                            preferred_element_type=jnp.float32)
    o_ref[...] = y


def _t1(x, e_flat, Wc, We):
    B = 2000
    return pl.pallas_call(
        _t1_body,
        grid=(N // B,),
        in_specs=[
            pl.BlockSpec((B, D), lambda i: (i, 0)),
            pl.BlockSpec((B, K * DE), lambda i: (i, 0)),
            pl.BlockSpec((D, D), lambda i: (0, 0)),
            pl.BlockSpec((D, DE), lambda i: (0, 0)),
        ],
        out_specs=pl.BlockSpec((B, D), lambda i: (i, 0)),
        out_shape=jax.ShapeDtypeStruct((N, D), jnp.float32),
    )(x, e_flat, Wc, We)


def _final_body(t1_ref, s_ref, wn_ref, o_ref):
    y = t1_ref[...] + lax.dot_general(
        s_ref[...] * (1.0 / K), wn_ref[...], _DOT11,
        preferred_element_type=jnp.float32)
    o_ref[...] = jnp.maximum(y, 0.0)


def _final(t1, s_pad, Wn):
    B = 2000
    return pl.pallas_call(
        _final_body,
        grid=(N // B,),
        in_specs=[
            pl.BlockSpec((B, D), lambda i: (i, 0)),
            pl.BlockSpec((B, D), lambda i: (i, 0)),
            pl.BlockSpec((D, D), lambda i: (0, 0)),
        ],
        out_specs=pl.BlockSpec((B, D), lambda i: (i, 0)),
        out_shape=jax.ShapeDtypeStruct((N, D), jnp.float32),
    )(t1, s_pad, Wn)


def kernel(x, e, ij, Wc, Wn, We):
    idx = ij.reshape(N * K)
    idx_pad = jnp.concatenate(
        [idx, jnp.zeros((NIP * K - N * K,), dtype=jnp.int32)])
    xp = _pack(x)
    s_pad = _sc_gather_sum(idx_pad, xp)
    t1 = _t1(x, e.reshape(N, K * DE), Wc, We)
    return _final(t1, s_pad, Wn)
